# baseline (speedup 1.0000x reference)
import functools

import jax
import jax.numpy as jnp
from jax import lax
from jax.experimental import pallas as pl
from jax.experimental.pallas import tpu as pltpu
from jax.experimental.pallas import tpu_sc as plsc

_K = 10
_D = 64
_B, _NC, _NT, _S, _F = 4, 4096, 1024, 3, 4
_N = _NC + _NT
_R = 256
_TILES = _N // _R
_CTX_TILES = _NC // _R
_E = _B * _K * _N
_NWORK = 32
_PW = _E // _NWORK
_CH = 800


def _ln(x):
    mu = jnp.mean(x, axis=-1, keepdims=True)
    var = jnp.mean((x - mu) ** 2, axis=-1, keepdims=True)
    return (x - mu) / jnp.sqrt(var + 1e-5)


def _bf(x):
    return x.astype(jnp.bfloat16)


def _knn_select(s_blk, ct, b):
    d2 = None
    for s in range(_S):
        diff = s_blk[:, s:s + 1] - ct[s:s + 1, :]
        sq = diff * diff
        d2 = sq if d2 is None else d2 + sq
    colf = lax.broadcasted_iota(jnp.int32, (_R, _NC), 1).astype(jnp.float32)
    idx_cols, dist_cols = [], []
    for k in range(_K):
        m = jnp.min(d2, axis=1, keepdims=True)
        ismin = d2 == m
        iv = jnp.min(jnp.where(ismin, colf, float(_NC)), axis=1, keepdims=True)
        dist_cols.append(jnp.sqrt(jnp.maximum(m, 0.0)))
        idx_cols.append(iv)
        if k + 1 < _K:
            d2 = jnp.where(colf == iv, jnp.inf, d2)
    idx = jnp.concatenate(idx_cols, axis=1).astype(jnp.int32) + b * _N
    dist = jnp.concatenate(dist_cols, axis=1)
    return idx, dist


def _proj(nd, wes_ref, wer_ref, be_ref, ps_ref, pc_ref):
    ndb = _bf(nd)
    ps_ref[...] = _bf(
        jnp.dot(ndb, wes_ref[...], preferred_element_type=jnp.float32))
    pc_ref[0] = (jnp.dot(ndb, wer_ref[...], preferred_element_type=jnp.float32)
                 + be_ref[...])


def _embed_body(s_ref, f_ref, ct_ref, tab_ref, w0a_ref, w0s_ref, w0f_ref,
                b0_ref, w1_ref, b1_ref, w2_ref, b2_ref, wes_ref, wer_ref,
                be_ref, idx_ref, dist_ref, out_ref, ps_ref, pc_ref):
    b = pl.program_id(0)
    t = pl.program_id(1)
    idx_ref[0], dist_ref[0] = _knn_select(s_ref[0], ct_ref[0], b)
    is_ctx = t < _CTX_TILES
    obs = jnp.where(is_ctx, tab_ref[1:2, :], tab_ref[0:1, :])
    obs_c = jnp.dot(_bf(obs), w0a_ref[...], preferred_element_type=jnp.float32)
    h = (jnp.dot(_bf(s_ref[0]), w0s_ref[...], preferred_element_type=jnp.float32)
         + jnp.dot(_bf(f_ref[0]), w0f_ref[...], preferred_element_type=jnp.float32)
         + obs_c + b0_ref[...])
    h = jax.nn.gelu(h)
    h = jax.nn.gelu(jnp.dot(_bf(h), w1_ref[...], preferred_element_type=jnp.float32)
                    + b1_ref[...])
    h = (jnp.dot(_bf(h), w2_ref[...], preferred_element_type=jnp.float32)
         + b2_ref[...])
    nd = _ln(h)
    out_ref[0] = nd
    _proj(nd, wes_ref, wer_ref, be_ref, ps_ref, pc_ref)


_PS_SPEC = pl.BlockSpec((_R, _D), lambda b, t: (b * _TILES + t, 0))
_PS_SHAPE = jax.ShapeDtypeStruct((_B * _N, _D), jnp.bfloat16)


def _embed(s_all, f_all, s_ctx_t, tab, w0a, w0s, w0f, b0, w1, b1, w2, b2,
           wes, wer, be):
    full = lambda *shape: pl.BlockSpec(shape, lambda b, t: tuple(0 for _ in shape))
    nd_spec = pl.BlockSpec((1, _R, _D), lambda b, t: (b, t, 0))
    k_spec = pl.BlockSpec((1, _R, _K), lambda b, t: (b, t, 0))
    return pl.pallas_call(
        _embed_body,
        grid=(_B, _TILES),
        in_specs=[
            pl.BlockSpec((1, _R, _S), lambda b, t: (b, t, 0)),
            pl.BlockSpec((1, _R, _F), lambda b, t: (b, t, 0)),
            pl.BlockSpec((1, _S, _NC), lambda b, t: (b, 0, 0)),
            full(2, 4), full(4, 256), full(_S, 256), full(_F, 256), full(256,),
            full(256, 128), full(128,), full(128, _D), full(_D,),
            full(_D, _D), full(_D, _D), full(_D,),
        ],
        out_specs=[k_spec, k_spec, nd_spec, _PS_SPEC, nd_spec],
        out_shape=[jax.ShapeDtypeStruct((_B, _N, _K), jnp.int32),
                   jax.ShapeDtypeStruct((_B, _N, _K), jnp.float32),
                   jax.ShapeDtypeStruct((_B, _N, _D), jnp.float32),
                   _PS_SHAPE,
                   jax.ShapeDtypeStruct((_B, _N, _D), jnp.float32)],
    )(s_all, f_all, s_ctx_t, tab, w0a, w0s, w0f, b0, w1, b1, w2, b2,
      wes, wer, be)


def _gather_sc_body(table_hbm, idx_hbm, out_hbm, idx_v, rows_v, gsem, wsem):
    wid = lax.axis_index("s") * 2 + lax.axis_index("c")
    base = wid * _PW
    nch = _PW // _CH
    pltpu.sync_copy(idx_hbm.at[pl.ds(base, _PW)], idx_v)

    def gather(c, buf):
        return pltpu.async_copy(
            table_hbm.at[idx_v.at[pl.ds(c * _CH, _CH)]], rows_v.at[buf], gsem)

    g = [None, None]
    w = [None, None]
    g[0] = gather(0, 0)
    for c in range(nch):
        b = c & 1
        nb = b ^ 1
        g[b].wait()
        if c + 1 < nch:
            if w[nb] is not None:
                w[nb].wait()
            g[nb] = gather(c + 1, nb)
        w[b] = pltpu.async_copy(
            rows_v.at[b], out_hbm.at[pl.ds(base + c * _CH, _CH)], wsem)
    w[nb].wait()
    w[b].wait()


def _gather_rows(table, idx_flat):
    return pl.kernel(
        _gather_sc_body,
        out_type=jax.ShapeDtypeStruct((_E, _D), jnp.bfloat16),
        mesh=plsc.VectorSubcoreMesh(core_axis_name="c", subcore_axis_name="s",
                                    num_cores=2, num_subcores=16),
        scratch_types=[
            pltpu.VMEM((_PW,), jnp.int32),
            pltpu.VMEM((2, _CH, _D), jnp.bfloat16),
            pltpu.SemaphoreType.DMA,
            pltpu.SemaphoreType.DMA,
        ],
        compiler_params=pltpu.CompilerParams(use_tc_tiling_on_sc=False),
    )(table, idx_flat)


def _post_body(*refs):
    g_refs = refs[:_K]
    (d_ref, pc_ref, nd_ref, wed_ref, wnt_ref, wnb_ref, bn_ref,
     wes_ref, wer_ref, be_ref, out_ref, ps_ref, pc_out_ref) = refs[_K:]
    c = pc_ref[0]
    dmat = _bf(d_ref[0]).astype(jnp.float32)
    wed = wed_ref[...]
    acc = None
    for k in range(_K):
        x = g_refs[k][...] + c + dmat[:, k:k + 1] * wed
        gx = jax.nn.gelu(x)
        acc = gx if acc is None else acc + gx
    nd = nd_ref[0]
    upd = jax.nn.gelu(
        jnp.dot(_bf(nd), wnt_ref[...], preferred_element_type=jnp.float32)
        + jnp.dot(_bf(acc), wnb_ref[...], preferred_element_type=jnp.float32)
        + bn_ref[...])
    nn = _ln(nd + upd)
    out_ref[0] = nn
    _proj(nn, wes_ref, wer_ref, be_ref, ps_ref, pc_out_ref)


def _post(g, dist, pc, nodes, we_d, wn_t, wn_b, bn, wes, wer, be):
    full = lambda *shape: pl.BlockSpec(shape, lambda b, t: tuple(0 for _ in shape))
    nd_spec = pl.BlockSpec((1, _R, _D), lambda b, t: (b, t, 0))
    nb = _TILES
    g_specs = [
        pl.BlockSpec((_R, _D),
                     functools.partial(
                         lambda k, b, t: (b * _K * nb + k * nb + t, 0), k))
        for k in range(_K)
    ]
    return pl.pallas_call(
        _post_body,
        grid=(_B, _TILES),
        in_specs=g_specs + [
            pl.BlockSpec((1, _R, _K), lambda b, t: (b, t, 0)),
            nd_spec, nd_spec,
            full(1, _D), full(_D, _D), full(_D, _D), full(_D,),
            full(_D, _D), full(_D, _D), full(_D,),
        ],
        out_specs=[nd_spec, _PS_SPEC, nd_spec],
        out_shape=[jax.ShapeDtypeStruct((_B, _N, _D), jnp.float32),
                   _PS_SHAPE,
                   jax.ShapeDtypeStruct((_B, _N, _D), jnp.float32)],
    )(*([g] * _K), dist, pc, nodes, we_d, wn_t, wn_b, bn, wes, wer, be)


def _head_body(x_ref, h0_ref, hb0_ref, h1_ref, hb1_ref, h2_ref, hb2_ref, o_ref):
    h = jax.nn.gelu(jnp.dot(_bf(x_ref[0]), h0_ref[...],
                            preferred_element_type=jnp.float32) + hb0_ref[...])
    h = jax.nn.gelu(jnp.dot(_bf(h), h1_ref[...],
                            preferred_element_type=jnp.float32) + hb1_ref[...])
    o_ref[0] = jnp.dot(_bf(h), h2_ref[...],
                       preferred_element_type=jnp.float32) + hb2_ref[...]


def _head(nodes, h0, hb0, h1, hb1, h2, hb2):
    full = lambda *shape: pl.BlockSpec(shape, lambda b, t: tuple(0 for _ in shape))
    return pl.pallas_call(
        _head_body,
        grid=(_B, _NT // _R),
        in_specs=[
            pl.BlockSpec((1, _R, _D), lambda b, t: (b, _CTX_TILES + t, 0)),
            full(_D, 256), full(256,), full(256, _D), full(_D,), full(_D, 2),
            full(2,),
        ],
        out_specs=pl.BlockSpec((1, _R, 2), lambda b, t: (b, t, 0)),
        out_shape=jax.ShapeDtypeStruct((_B, _NT, 2), jnp.float32),
    )(nodes, h0, hb0, h1, hb1, h2, hb2)


def kernel(s_ctx, f_ctx, s_test, embed_obs_table, W0, b0, W1, b1, W2, b2,
           blk_We, blk_be, blk_Wn, blk_bn, H0, hb0, H1, hb1, H2, hb2):
    s_all = jnp.concatenate([s_ctx, s_test], axis=1)
    f_all = jnp.concatenate(
        [f_ctx, jnp.zeros((_B, _NT, _F), f_ctx.dtype)], axis=1)
    s_ctx_t = s_ctx.transpose(0, 2, 1)

    bf = _bf
    we_s = bf(blk_We[:, :_D])
    we_r = bf(blk_We[:, _D:2 * _D])
    we_d = bf(blk_We[:, 2 * _D:]).astype(jnp.float32)
    wn_t = bf(blk_Wn[:, :_D])
    wn_b = bf(blk_Wn[:, _D:])

    idx_g, dist, nodes, ps, pc = _embed(
        s_all, f_all, s_ctx_t, bf(embed_obs_table),
        bf(W0[0:4]), bf(W0[4:4 + _S]), bf(W0[4 + _S:]), b0,
        bf(W1), b1, bf(W2), b2,
        we_s[0], we_r[0], blk_be[0])
    idx_flat = idx_g.transpose(0, 2, 1).reshape(_E)

    nblk = blk_We.shape[0]
    for i in range(nblk):
        g = _gather_rows(ps, idx_flat)
        j = min(i + 1, nblk - 1)
        nodes, ps, pc = _post(g, dist, pc, nodes,
                              we_d[i], wn_t[i], wn_b[i], blk_bn[i],
                              we_s[j], we_r[j], blk_be[j])

    out = _head(nodes, bf(H0), hb0, bf(H1), hb1, bf(H2), hb2)
    return out[..., 0], out[..., 1]


# speedup vs baseline: 1.1260x; 1.1260x over previous
import functools

import jax
import jax.numpy as jnp
from jax import lax
from jax.experimental import pallas as pl
from jax.experimental.pallas import tpu as pltpu
from jax.experimental.pallas import tpu_sc as plsc

_K = 10
_D = 64
_B, _NC, _NT, _S, _F = 4, 4096, 1024, 3, 4
_N = _NC + _NT
_R = 256
_TILES = _N // _R
_CTX_TILES = _NC // _R
_E = _B * _K * _N
_NWORK = 32
_PW = _E // _NWORK
_CH = 800


def _ln(x):
    mu = jnp.mean(x, axis=-1, keepdims=True)
    var = jnp.mean((x - mu) ** 2, axis=-1, keepdims=True)
    return (x - mu) / jnp.sqrt(var + 1e-5)


def _bf(x):
    return x.astype(jnp.bfloat16)


def _knn_select(s_blk, ct, b):
    d2 = None
    for s in range(_S):
        diff = s_blk[:, s:s + 1] - ct[s:s + 1, :]
        sq = diff * diff
        d2 = sq if d2 is None else d2 + sq
    colf = lax.broadcasted_iota(jnp.int32, (_R, _NC), 1).astype(jnp.float32)
    idx_cols, dist_cols = [], []
    for k in range(_K):
        m = jnp.min(d2, axis=1, keepdims=True)
        ismin = d2 == m
        iv = jnp.min(jnp.where(ismin, colf, float(_NC)), axis=1, keepdims=True)
        dist_cols.append(jnp.sqrt(jnp.maximum(m, 0.0)))
        idx_cols.append(iv)
        if k + 1 < _K:
            d2 = jnp.where(colf == iv, jnp.inf, d2)
    idx = jnp.concatenate(idx_cols, axis=1).astype(jnp.int32) + b * _N
    dist = jnp.concatenate(dist_cols, axis=1)
    return idx, dist


def _proj(nd, wes_ref, wer_ref, be_ref, ps_ref, pc_ref):
    ndb = _bf(nd)
    ps_ref[...] = jnp.dot(ndb, wes_ref[...], preferred_element_type=jnp.float32)
    pc_ref[0] = (jnp.dot(ndb, wer_ref[...], preferred_element_type=jnp.float32)
                 + be_ref[...])


def _embed_body(s_ref, f_ref, ct_ref, tab_ref, w0a_ref, w0s_ref, w0f_ref,
                b0_ref, w1_ref, b1_ref, w2_ref, b2_ref, wes_ref, wer_ref,
                be_ref, idx_ref, dist_ref, out_ref, ps_ref, pc_ref):
    b = pl.program_id(0)
    t = pl.program_id(1)
    idx_ref[0], dist_ref[0] = _knn_select(s_ref[0], ct_ref[0], b)
    is_ctx = t < _CTX_TILES
    obs = jnp.where(is_ctx, tab_ref[1:2, :], tab_ref[0:1, :])
    obs_c = jnp.dot(_bf(obs), w0a_ref[...], preferred_element_type=jnp.float32)
    h = (jnp.dot(_bf(s_ref[0]), w0s_ref[...], preferred_element_type=jnp.float32)
         + jnp.dot(_bf(f_ref[0]), w0f_ref[...], preferred_element_type=jnp.float32)
         + obs_c + b0_ref[...])
    h = jax.nn.gelu(h)
    h = jax.nn.gelu(jnp.dot(_bf(h), w1_ref[...], preferred_element_type=jnp.float32)
                    + b1_ref[...])
    h = (jnp.dot(_bf(h), w2_ref[...], preferred_element_type=jnp.float32)
         + b2_ref[...])
    nd = _ln(h)
    out_ref[0] = nd
    _proj(nd, wes_ref, wer_ref, be_ref, ps_ref, pc_ref)


_PS_SPEC = pl.BlockSpec((_R, _D), lambda b, t: (b * _TILES + t, 0))
_PS_SHAPE = jax.ShapeDtypeStruct((_B * _N, _D), jnp.float32)


def _embed(s_all, f_all, s_ctx_t, tab, w0a, w0s, w0f, b0, w1, b1, w2, b2,
           wes, wer, be):
    full = lambda *shape: pl.BlockSpec(shape, lambda b, t: tuple(0 for _ in shape))
    nd_spec = pl.BlockSpec((1, _R, _D), lambda b, t: (b, t, 0))
    k_spec = pl.BlockSpec((1, _R, _K), lambda b, t: (b, t, 0))
    return pl.pallas_call(
        _embed_body,
        grid=(_B, _TILES),
        in_specs=[
            pl.BlockSpec((1, _R, _S), lambda b, t: (b, t, 0)),
            pl.BlockSpec((1, _R, _F), lambda b, t: (b, t, 0)),
            pl.BlockSpec((1, _S, _NC), lambda b, t: (b, 0, 0)),
            full(2, 4), full(4, 256), full(_S, 256), full(_F, 256), full(256,),
            full(256, 128), full(128,), full(128, _D), full(_D,),
            full(_D, _D), full(_D, _D), full(_D,),
        ],
        out_specs=[k_spec, k_spec, nd_spec, _PS_SPEC, nd_spec],
        out_shape=[jax.ShapeDtypeStruct((_B, _N, _K), jnp.int32),
                   jax.ShapeDtypeStruct((_B, _N, _K), jnp.float32),
                   jax.ShapeDtypeStruct((_B, _N, _D), jnp.float32),
                   _PS_SHAPE,
                   jax.ShapeDtypeStruct((_B, _N, _D), jnp.float32)],
    )(s_all, f_all, s_ctx_t, tab, w0a, w0s, w0f, b0, w1, b1, w2, b2,
      wes, wer, be)


def _gather_sc_body(pw, table_hbm, idx_hbm, out_hbm, idx_v, rows_v, gsem, wsem):
    wid = lax.axis_index("s") * 2 + lax.axis_index("c")
    base = wid * pw
    nch = pw // _CH
    pltpu.sync_copy(idx_hbm.at[pl.ds(base, pw)], idx_v)

    def gather(c, buf):
        return pltpu.async_copy(
            table_hbm.at[idx_v.at[pl.ds(c * _CH, _CH)]], rows_v.at[buf], gsem)

    g = [None, None]
    w = [None, None]
    g[0] = gather(0, 0)
    for c in range(nch):
        b = c & 1
        nb = b ^ 1
        g[b].wait()
        if c + 1 < nch:
            if w[nb] is not None:
                w[nb].wait()
            g[nb] = gather(c + 1, nb)
        w[b] = pltpu.async_copy(
            rows_v.at[b], out_hbm.at[pl.ds(base + c * _CH, _CH)], wsem)
    w[nb].wait()
    w[b].wait()


def _gather_rows(table, idx_flat):
    e = idx_flat.shape[0]
    pw = e // _NWORK
    return pl.kernel(
        functools.partial(_gather_sc_body, pw),
        out_type=jax.ShapeDtypeStruct((e, _D), jnp.float32),
        mesh=plsc.VectorSubcoreMesh(core_axis_name="c", subcore_axis_name="s",
                                    num_cores=2, num_subcores=16),
        scratch_types=[
            pltpu.VMEM((pw,), jnp.int32),
            pltpu.VMEM((2, _CH, _D), jnp.float32),
            pltpu.SemaphoreType.DMA,
            pltpu.SemaphoreType.DMA,
        ],
        compiler_params=pltpu.CompilerParams(use_tc_tiling_on_sc=False),
    )(table, idx_flat)


def _post_body(*refs):
    g_refs = refs[:_K]
    (d_ref, pc_ref, nd_ref, wed_ref, wnt_ref, wnb_ref, bn_ref,
     wes_ref, wer_ref, be_ref, out_ref, ps_ref, pc_out_ref) = refs[_K:]
    c = pc_ref[0]
    dmat = _bf(d_ref[0]).astype(jnp.float32)
    wed = wed_ref[...]
    acc = None
    for k in range(_K):
        x = g_refs[k][...] + c + dmat[:, k:k + 1] * wed
        gx = jax.nn.gelu(x)
        acc = gx if acc is None else acc + gx
    nd = nd_ref[0]
    upd = jax.nn.gelu(
        jnp.dot(_bf(nd), wnt_ref[...], preferred_element_type=jnp.float32)
        + jnp.dot(_bf(acc), wnb_ref[...], preferred_element_type=jnp.float32)
        + bn_ref[...])
    nn = _ln(nd + upd)
    out_ref[0] = nn
    _proj(nn, wes_ref, wer_ref, be_ref, ps_ref, pc_out_ref)


_HB = 2
_EH = _E // 2


def _post(off, g, dist, pc, nodes, we_d, wn_t, wn_b, bn, wes, wer, be):
    full = lambda *shape: pl.BlockSpec(shape, lambda b, t: tuple(0 for _ in shape))
    nd_spec = pl.BlockSpec((1, _R, _D), lambda b, t: (b, t, 0))
    nb = _TILES
    g_specs = [
        pl.BlockSpec((_R, _D),
                     functools.partial(
                         lambda k, b, t: (b * _K * nb + k * nb + t, 0), k))
        for k in range(_K)
    ]
    ps_spec = pl.BlockSpec((_R, _D), lambda b, t: (b * _TILES + t, 0))
    return pl.pallas_call(
        _post_body,
        grid=(_HB, _TILES),
        in_specs=g_specs + [
            pl.BlockSpec((1, _R, _K),
                         functools.partial(
                             lambda o, b, t: (o + b, t, 0), off)),
            nd_spec, nd_spec,
            full(1, _D), full(_D, _D), full(_D, _D), full(_D,),
            full(_D, _D), full(_D, _D), full(_D,),
        ],
        out_specs=[nd_spec, ps_spec, nd_spec],
        out_shape=[jax.ShapeDtypeStruct((_HB, _N, _D), jnp.float32),
                   jax.ShapeDtypeStruct((_HB * _N, _D), jnp.float32),
                   jax.ShapeDtypeStruct((_HB, _N, _D), jnp.float32)],
    )(*([g] * _K), dist, pc, nodes, we_d, wn_t, wn_b, bn, wes, wer, be)


def _head_body(x_ref, h0_ref, hb0_ref, h1_ref, hb1_ref, h2_ref, hb2_ref, o_ref):
    h = jax.nn.gelu(jnp.dot(_bf(x_ref[0]), h0_ref[...],
                            preferred_element_type=jnp.float32) + hb0_ref[...])
    h = jax.nn.gelu(jnp.dot(_bf(h), h1_ref[...],
                            preferred_element_type=jnp.float32) + hb1_ref[...])
    o_ref[0] = jnp.dot(_bf(h), h2_ref[...],
                       preferred_element_type=jnp.float32) + hb2_ref[...]


def _head(nodes, h0, hb0, h1, hb1, h2, hb2):
    full = lambda *shape: pl.BlockSpec(shape, lambda b, t: tuple(0 for _ in shape))
    return pl.pallas_call(
        _head_body,
        grid=(_HB, _NT // _R),
        in_specs=[
            pl.BlockSpec((1, _R, _D), lambda b, t: (b, _CTX_TILES + t, 0)),
            full(_D, 256), full(256,), full(256, _D), full(_D,), full(_D, 2),
            full(2,),
        ],
        out_specs=pl.BlockSpec((1, _R, 2), lambda b, t: (b, t, 0)),
        out_shape=jax.ShapeDtypeStruct((_HB, _NT, 2), jnp.float32),
    )(nodes, h0, hb0, h1, hb1, h2, hb2)


def kernel(s_ctx, f_ctx, s_test, embed_obs_table, W0, b0, W1, b1, W2, b2,
           blk_We, blk_be, blk_Wn, blk_bn, H0, hb0, H1, hb1, H2, hb2):
    s_all = jnp.concatenate([s_ctx, s_test], axis=1)
    f_all = jnp.concatenate(
        [f_ctx, jnp.zeros((_B, _NT, _F), f_ctx.dtype)], axis=1)
    s_ctx_t = s_ctx.transpose(0, 2, 1)

    bf = _bf
    we_s = bf(blk_We[:, :_D])
    we_r = bf(blk_We[:, _D:2 * _D])
    we_d = bf(blk_We[:, 2 * _D:]).astype(jnp.float32)
    wn_t = bf(blk_Wn[:, :_D])
    wn_b = bf(blk_Wn[:, _D:])

    idx_g, dist, nodes, ps, pc = _embed(
        s_all, f_all, s_ctx_t, bf(embed_obs_table),
        bf(W0[0:4]), bf(W0[4:4 + _S]), bf(W0[4 + _S:]), b0,
        bf(W1), b1, bf(W2), b2,
        we_s[0], we_r[0], blk_be[0])
    idx_flat = idx_g.transpose(0, 2, 1).reshape(_E)
    idx_h = [idx_flat[:_EH], idx_flat[_EH:]]
    idx_h_local = [idx_h[0], idx_h[1] - _HB * _N]

    nodes_h = [nodes[:_HB], nodes[_HB:]]
    pc_h = [pc[:_HB], pc[_HB:]]
    ps_h = [ps, ps]
    idx0 = [idx_h[0], idx_h[1]]
    nblk = blk_We.shape[0]
    for i in range(nblk):
        j = min(i + 1, nblk - 1)
        for h in range(2):
            idx = idx0[h] if i == 0 else idx_h_local[h]
            g = _gather_rows(ps_h[h], idx)
            nodes_h[h], ps_h[h], pc_h[h] = _post(
                h * _HB, g, dist, pc_h[h], nodes_h[h],
                we_d[i], wn_t[i], wn_b[i], blk_bn[i],
                we_s[j], we_r[j], blk_be[j])

    out = jnp.concatenate(
        [_head(nodes_h[h], bf(H0), hb0, bf(H1), hb1, bf(H2), hb2)
         for h in range(2)], axis=0)
    return out[..., 0], out[..., 1]
